# Initial kernel scaffold; baseline (speedup 1.0000x reference)
import functools

import jax
import jax.numpy as jnp
from jax import lax
from jax.experimental import pallas as pl
from jax.experimental.pallas import tpu as pltpu
from jax.experimental.pallas import tpu_sc as plsc

N = 10000
D = 128
E = 320000

NC = 2
NS = 16
NW = NC * NS
CH = 128
E_PAD = ((E + NW * CH - 1) // (NW * CH)) * (NW * CH)
EPW = E_PAD // NW
NCHUNK = EPW // CH
N_ACC = 10240
RPT = N_ACC // NS
CW = 16

_mesh = plsc.VectorSubcoreMesh(
    core_axis_name="c", subcore_axis_name="s", num_cores=NC, num_subcores=NS)


def _sc_agg_body(with_cnt, *refs):
    if with_cnt:
        (table, src, dst, zrow, zcnt, ones_h,
         parts, cparts,
         sidx, didx, rows, ones_v, acc, cacc, sem) = refs
    else:
        (table, src, dst, zrow,
         parts,
         sidx, didx, rows, acc, sem) = refs

    c = lax.axis_index("c")
    s = lax.axis_index("s")
    w = s * NC + c

    pltpu.sync_copy(zrow, acc.at[pl.ds(s * RPT, RPT)])
    if with_cnt:
        pltpu.sync_copy(zcnt, cacc.at[pl.ds(s * RPT, RPT)])
        pltpu.sync_copy(ones_h, ones_v)
    plsc.subcore_barrier()

    def chunk(j, carry):
        base = pl.multiple_of((w * NCHUNK + j) * CH, CH)
        pltpu.sync_copy(src.at[pl.ds(base, CH)], sidx)
        pltpu.sync_copy(dst.at[pl.ds(base, CH)], didx)
        pltpu.async_copy(table.at[sidx], rows, sem).wait()
        pltpu.sync_copy(rows, acc.at[didx], add=True)
        if with_cnt:
            pltpu.sync_copy(ones_v, cacc.at[didx], add=True)
        return carry

    lax.fori_loop(0, NCHUNK, chunk, 0)
    plsc.subcore_barrier()

    pltpu.sync_copy(acc.at[pl.ds(s * RPT, RPT)],
                    parts.at[c, pl.ds(s * RPT, RPT)])
    if with_cnt:
        pltpu.sync_copy(cacc.at[pl.ds(s * RPT, RPT)],
                        cparts.at[c, pl.ds(s * RPT, RPT)])


def _sc_agg(table, src, dst, with_cnt):
    zrow = jnp.zeros((RPT, D), jnp.float32)
    out_type = [jax.ShapeDtypeStruct((NC, N_ACC, D), jnp.float32)]
    scratch = [
        pltpu.VMEM((CH,), jnp.int32),
        pltpu.VMEM((CH,), jnp.int32),
        pltpu.VMEM((CH, D), jnp.float32),
    ]
    if with_cnt:
        out_type.append(jax.ShapeDtypeStruct((NC, N_ACC, CW), jnp.float32))
        scratch.append(pltpu.VMEM((CH, CW), jnp.float32))
    scratch.append(pltpu.VMEM_SHARED((N_ACC, D), jnp.float32))
    if with_cnt:
        scratch.append(pltpu.VMEM_SHARED((N_ACC, CW), jnp.float32))
    scratch.append(pltpu.SemaphoreType.DMA)

    kern = pl.kernel(
        functools.partial(_sc_agg_body, with_cnt),
        out_type=out_type,
        mesh=_mesh,
        scratch_types=scratch,
    )
    if with_cnt:
        zcnt = jnp.zeros((RPT, CW), jnp.float32)
        ones_h = jnp.ones((CH, CW), jnp.float32)
        return kern(table, src, dst, zrow, zcnt, ones_h)
    return kern(table, src, dst, zrow)[0]


def _tc1_body(p0, p1, c0, c1, x, wl, b, wr, out, inv_out):
    agg = p0[0] + p1[0]
    cnt = c0[0][:, :1] + c1[0][:, :1]
    inv = 1.0 / jnp.maximum(cnt, 1.0)
    mean = agg * inv
    y = (jnp.dot(mean, wl[...], preferred_element_type=jnp.float32)
         + jnp.dot(x[...], wr[...], preferred_element_type=jnp.float32)
         + b[...])
    out[...] = jnp.maximum(y, 0.0)
    inv_out[...] = jnp.broadcast_to(inv, inv_out.shape)


def _tc2_body(p0, p1, inv_in, x, wl, b, wr, out):
    agg = p0[0] + p1[0]
    inv = inv_in[:, :1]
    mean = agg * inv
    out[...] = (jnp.dot(mean, wl[...], preferred_element_type=jnp.float32)
                + jnp.dot(x[...], wr[...], preferred_element_type=jnp.float32)
                + b[...])


_RB = 1000
_NB = N // _RB


def _tc_layer1(parts, cparts, x, wl, b, wr):
    return pl.pallas_call(
        _tc1_body,
        grid=(_NB,),
        in_specs=[
            pl.BlockSpec((1, _RB, D), lambda i: (0, i, 0)),
            pl.BlockSpec((1, _RB, D), lambda i: (1, i, 0)),
            pl.BlockSpec((1, _RB, CW), lambda i: (0, i, 0)),
            pl.BlockSpec((1, _RB, CW), lambda i: (1, i, 0)),
            pl.BlockSpec((_RB, D), lambda i: (i, 0)),
            pl.BlockSpec((D, D), lambda i: (0, 0)),
            pl.BlockSpec((1, D), lambda i: (0, 0)),
            pl.BlockSpec((D, D), lambda i: (0, 0)),
        ],
        out_specs=[
            pl.BlockSpec((_RB, D), lambda i: (i, 0)),
            pl.BlockSpec((_RB, 8), lambda i: (i, 0)),
        ],
        out_shape=[
            jax.ShapeDtypeStruct((N, D), jnp.float32),
            jax.ShapeDtypeStruct((N, 8), jnp.float32),
        ],
    )(parts, parts, cparts, cparts, x, wl, b, wr)


def _tc_layer2(parts, inv, x, wl, b, wr):
    return pl.pallas_call(
        _tc2_body,
        grid=(_NB,),
        in_specs=[
            pl.BlockSpec((1, _RB, D), lambda i: (0, i, 0)),
            pl.BlockSpec((1, _RB, D), lambda i: (1, i, 0)),
            pl.BlockSpec((_RB, 8), lambda i: (i, 0)),
            pl.BlockSpec((_RB, D), lambda i: (i, 0)),
            pl.BlockSpec((D, D), lambda i: (0, 0)),
            pl.BlockSpec((1, D), lambda i: (0, 0)),
            pl.BlockSpec((D, D), lambda i: (0, 0)),
        ],
        out_specs=pl.BlockSpec((_RB, D), lambda i: (i, 0)),
        out_shape=jax.ShapeDtypeStruct((N, D), jnp.float32),
    )(parts, parts, inv, x, wl, b, wr)


@jax.jit
def kernel(x, edge_index, W1_l, b1_l, W1_r, W2_l, b2_l, W2_r):
    src = edge_index[0]
    dst = edge_index[1]
    npad = E_PAD - E
    src_p = jnp.concatenate([src, jnp.zeros((npad,), jnp.int32)])
    dst_p = jnp.concatenate([dst, jnp.full((npad,), N, jnp.int32)])

    parts1, cparts = _sc_agg(x, src_p, dst_p, with_cnt=True)
    h, inv = _tc_layer1(parts1, cparts, x, W1_l, b1_l.reshape(1, D), W1_r)
    parts2 = _sc_agg(h, src_p, dst_p, with_cnt=False)
    out = _tc_layer2(parts2, inv, h, W2_l, b2_l.reshape(1, D), W2_r)
    return out


# speedup vs baseline: 4.2411x; 4.2411x over previous
import functools

import jax
import jax.numpy as jnp
from jax import lax
from jax.experimental import pallas as pl
from jax.experimental.pallas import tpu as pltpu
from jax.experimental.pallas import tpu_sc as plsc

N = 10000
D = 128
E = 320000

NC = 2
NS = 16
NW = NC * NS
L = 16
CH = 128
E_PAD = ((E + NW * CH - 1) // (NW * CH)) * (NW * CH)
EPW = E_PAD // NW
NCHUNK = EPW // CH
N_ACC = 10240
RPT = N_ACC // NS

_mesh = plsc.VectorSubcoreMesh(
    core_axis_name="c", subcore_axis_name="s", num_cores=NC, num_subcores=NS)


def _sc_agg_body(with_cnt, *refs):
    if with_cnt:
        (table, src, dst, zrow, zhist,
         parts, chist,
         sidx, didx, rows, hist, acc, sem) = refs
    else:
        (table, src, dst, zrow,
         parts,
         sidx, didx, rows, acc, sem) = refs

    c = lax.axis_index("c")
    s = lax.axis_index("s")
    w = s * NC + c

    pltpu.sync_copy(zrow, rows)
    for k in range(RPT // CH):
        pltpu.sync_copy(rows, acc.at[pl.ds(s * RPT + k * CH, CH)])
    if with_cnt:
        pltpu.sync_copy(zhist, hist)
    plsc.subcore_barrier()

    ones16 = jnp.ones((L,), jnp.float32)

    def chunk(j, carry):
        base = pl.multiple_of((w * NCHUNK + j) * CH, CH)
        pltpu.sync_copy(src.at[pl.ds(base, CH)], sidx)
        pltpu.sync_copy(dst.at[pl.ds(base, CH)], didx)
        pltpu.async_copy(table.at[sidx], rows, sem).wait()
        pltpu.sync_copy(rows, acc.at[didx], add=True)
        if with_cnt:
            for jj in range(CH // L):
                idx16 = didx[pl.ds(jj * L, L)]
                plsc.addupdate_scatter(hist, [idx16], ones16)
        return carry

    lax.fori_loop(0, NCHUNK, chunk, 0)
    plsc.subcore_barrier()

    for k in range(RPT // CH):
        pltpu.sync_copy(acc.at[pl.ds(s * RPT + k * CH, CH)], rows)
        pltpu.sync_copy(rows, parts.at[c, pl.ds(s * RPT + k * CH, CH)])
    if with_cnt:
        pltpu.sync_copy(hist, chist.at[w])


def _sc_agg(table, src, dst, with_cnt):
    zrow = jnp.zeros((CH, D), jnp.float32)
    out_type = [jax.ShapeDtypeStruct((NC, N_ACC, D), jnp.float32)]
    scratch = [
        pltpu.VMEM((CH,), jnp.int32),
        pltpu.VMEM((CH,), jnp.int32),
        pltpu.VMEM((CH, D), jnp.float32),
    ]
    if with_cnt:
        out_type.append(jax.ShapeDtypeStruct((NW, N_ACC), jnp.float32))
        scratch.append(pltpu.VMEM((N_ACC,), jnp.float32))
    scratch.append(pltpu.VMEM_SHARED((N_ACC, D), jnp.float32))
    scratch.append(pltpu.SemaphoreType.DMA)

    kern = pl.kernel(
        functools.partial(_sc_agg_body, with_cnt),
        out_type=out_type,
        mesh=_mesh,
        scratch_types=scratch,
        compiler_params=pltpu.CompilerParams(needs_layout_passes=False),
    )
    if with_cnt:
        zhist = jnp.zeros((N_ACC,), jnp.float32)
        return kern(table, src, dst, zrow, zhist)
    return kern(table, src, dst, zrow)[0]


def _tc1_body(p0, p1, ch, x, wl, b, wr, out, inv_out):
    agg = p0[0] + p1[0]
    cnt = jnp.sum(ch[...], axis=0)[:, None]
    inv = 1.0 / jnp.maximum(cnt, 1.0)
    mean = agg * inv
    y = (jnp.dot(mean, wl[...], preferred_element_type=jnp.float32)
         + jnp.dot(x[...], wr[...], preferred_element_type=jnp.float32)
         + b[...])
    out[...] = jnp.maximum(y, 0.0)
    inv_out[...] = jnp.broadcast_to(inv, inv_out.shape)


def _tc2_body(p0, p1, inv_in, x, wl, b, wr, out):
    agg = p0[0] + p1[0]
    inv = inv_in[:, :1]
    mean = agg * inv
    out[...] = (jnp.dot(mean, wl[...], preferred_element_type=jnp.float32)
                + jnp.dot(x[...], wr[...], preferred_element_type=jnp.float32)
                + b[...])


_RB = 1024
_NB = N_ACC // _RB


def _tc_layer1(parts, chist, x, wl, b, wr):
    return pl.pallas_call(
        _tc1_body,
        grid=(_NB,),
        in_specs=[
            pl.BlockSpec((1, _RB, D), lambda i: (0, i, 0)),
            pl.BlockSpec((1, _RB, D), lambda i: (1, i, 0)),
            pl.BlockSpec((NW, _RB), lambda i: (0, i)),
            pl.BlockSpec((_RB, D), lambda i: (i, 0)),
            pl.BlockSpec((D, D), lambda i: (0, 0)),
            pl.BlockSpec((1, D), lambda i: (0, 0)),
            pl.BlockSpec((D, D), lambda i: (0, 0)),
        ],
        out_specs=[
            pl.BlockSpec((_RB, D), lambda i: (i, 0)),
            pl.BlockSpec((_RB, 8), lambda i: (i, 0)),
        ],
        out_shape=[
            jax.ShapeDtypeStruct((N_ACC, D), jnp.float32),
            jax.ShapeDtypeStruct((N_ACC, 8), jnp.float32),
        ],
    )(parts, parts, chist, x, wl, b, wr)


def _tc_layer2(parts, inv, x, wl, b, wr):
    return pl.pallas_call(
        _tc2_body,
        grid=(_NB,),
        in_specs=[
            pl.BlockSpec((1, _RB, D), lambda i: (0, i, 0)),
            pl.BlockSpec((1, _RB, D), lambda i: (1, i, 0)),
            pl.BlockSpec((_RB, 8), lambda i: (i, 0)),
            pl.BlockSpec((_RB, D), lambda i: (i, 0)),
            pl.BlockSpec((D, D), lambda i: (0, 0)),
            pl.BlockSpec((1, D), lambda i: (0, 0)),
            pl.BlockSpec((D, D), lambda i: (0, 0)),
        ],
        out_specs=pl.BlockSpec((_RB, D), lambda i: (i, 0)),
        out_shape=jax.ShapeDtypeStruct((N_ACC, D), jnp.float32),
    )(parts, parts, inv, x, wl, b, wr)


@jax.jit
def kernel(x, edge_index, W1_l, b1_l, W1_r, W2_l, b2_l, W2_r):
    src = edge_index[0]
    dst = edge_index[1]
    npad = E_PAD - E
    src_p = jnp.concatenate([src, jnp.zeros((npad,), jnp.int32)])
    dst_p = jnp.concatenate([dst, jnp.full((npad,), N, jnp.int32)])
    xp = jnp.pad(x, ((0, N_ACC - N), (0, 0)))

    parts1, chist = _sc_agg(xp, src_p, dst_p, with_cnt=True)
    h, inv = _tc_layer1(parts1, chist, xp, W1_l, b1_l.reshape(1, D), W1_r)
    parts2 = _sc_agg(h, src_p, dst_p, with_cnt=False)
    out = _tc_layer2(parts2, inv, h, W2_l, b2_l.reshape(1, D), W2_r)
    return out[:N]
